# Initial kernel scaffold; baseline (speedup 1.0000x reference)
import jax
import jax.numpy as jnp
from jax.experimental import pallas as pl


def kernel(input_ids, embedding_weight):
    raise NotImplementedError("write your pallas kernel here")


# speedup vs baseline: 3.5337x; 3.5337x over previous
import functools

import jax
import jax.numpy as jnp
from jax import lax
from jax.experimental import pallas as pl
from jax.experimental.pallas import tpu as pltpu
from jax.experimental.pallas import tpu_sc as plsc

VOCAB = 64
HID = 8
BATCH = 32
SEQ = 8192
NTOK = BATCH * SEQ
NW = 32
TOK_PER_W = NTOK // NW
CHUNK = 512
NCHUNK = TOK_PER_W // CHUNK
ROWS = CHUNK // 128
IDS_ROWS = NTOK // 128

_C23 = 8388608.0


def _sc_body(ids_hbm, w2d_hbm, logits_hbm, hidden_hbm,
             ids_v, hid_v, log_buf, idx_save, w_v, idx_tab, val_tab,
             sem):
    wid = lax.axis_index("s") * 2 + lax.axis_index("c")
    iota = lax.iota(jnp.int32, 16)
    zf = jnp.zeros((16,), jnp.float32)

    pltpu.sync_copy(w2d_hbm, w_v)
    for vg in range(VOCAB // 16):
        v16 = iota + vg * 16
        w0 = plsc.load_gather(w_v, [v16, iota * 0])
        t = (w0 * 10.0 + _C23) - _C23
        t = jnp.maximum(t, 0.0)
        i16 = lax.bitwise_and(t.astype(jnp.int32), VOCAB - 1)
        v16f = i16.astype(jnp.float32) / 10.0
        idx_tab[pl.ds(vg * 16, 16)] = i16
        val_tab[pl.ds(vg * 16, 16)] = v16f

    def zero_body(i, carry):
        for q in range(VOCAB // 16):
            log_buf[i, pl.ds(q * 16, 16)] = zf
        return carry

    lax.fori_loop(0, CHUNK, zero_body, None)

    def chunk_body(c, carry):
        base_row = wid * (TOK_PER_W // 128) + c * ROWS
        tok0 = wid * TOK_PER_W + c * CHUNK
        pltpu.sync_copy(ids_hbm.at[pl.ds(base_row, ROWS)], ids_v)
        cps = [pltpu.async_copy(w2d_hbm.at[ids_v.at[j]], hid_v.at[j], sem)
               for j in range(ROWS)]
        for g in range(CHUNK // 16):
            j, o = divmod(g, 8)
            ids16 = ids_v[j, pl.ds(o * 16, 16)]
            i16 = plsc.load_gather(idx_tab, [ids16])
            v16 = plsc.load_gather(val_tab, [ids16])
            row16 = iota + g * 16
            plsc.store_scatter(log_buf, [row16, i16], v16)
            idx_save[pl.ds(g * 16, 16)] = i16
        for cp in cps:
            cp.wait()
        pltpu.sync_copy(hid_v, hidden_hbm.at[pl.ds(base_row, ROWS)])
        pltpu.sync_copy(log_buf, logits_hbm.at[pl.ds(tok0, CHUNK)])
        for g in range(CHUNK // 16):
            row16 = iota + g * 16
            i16 = idx_save[pl.ds(g * 16, 16)]
            plsc.store_scatter(log_buf, [row16, i16], zf)
        return carry

    lax.fori_loop(0, NCHUNK, chunk_body, None)


@functools.partial(
    pl.kernel,
    out_type=[
        jax.ShapeDtypeStruct((NTOK, VOCAB), jnp.float32),
        jax.ShapeDtypeStruct((IDS_ROWS, 128, HID), jnp.float32),
    ],
    mesh=plsc.VectorSubcoreMesh(core_axis_name="c", subcore_axis_name="s"),
    compiler_params=pltpu.CompilerParams(
        needs_layout_passes=False, use_tc_tiling_on_sc=False),
    scratch_types=[
        pltpu.VMEM((ROWS, 128), jnp.int32),
        pltpu.VMEM((ROWS, 128, HID), jnp.float32),
        pltpu.VMEM((CHUNK, VOCAB), jnp.float32),
        pltpu.VMEM((CHUNK,), jnp.int32),
        pltpu.VMEM((VOCAB, HID), jnp.float32),
        pltpu.VMEM((VOCAB,), jnp.int32),
        pltpu.VMEM((VOCAB,), jnp.float32),
        pltpu.SemaphoreType.DMA,
    ],
)
def _fake_model_sc(*refs):
    _sc_body(*refs)


def kernel(input_ids, embedding_weight):
    ids = input_ids.astype(jnp.int32).reshape(IDS_ROWS, 128)
    w = embedding_weight.astype(jnp.float32)
    logits_flat, hidden3 = _fake_model_sc(ids, w)
    return (logits_flat.reshape(BATCH, SEQ, VOCAB),
            hidden3.reshape(BATCH, SEQ, HID))
